# baseline (speedup 1.0000x reference)
import functools

import jax
import jax.numpy as jnp
from jax.experimental import pallas as pl

_W = 5.0
_GAMMA = 0.02


def _nn_kernel(x_ref, rhs_ref, nc_ref, out_ref, *, nt):
    t = pl.program_id(1)

    x = x_ref[0]
    rhs = rhs_ref[0]
    nc = nc_ref[0]

    d = jax.lax.dot_general(
        x, rhs, (((1,), (0,)), ((), ())),
        preferred_element_type=jnp.float32,
    )

    m = jnp.min(d, axis=1, keepdims=True)
    mask = jnp.where(d == m, 1.0, 0.0).astype(jnp.bfloat16)
    g = jax.lax.dot_general(
        mask, nc, (((1,), (0,)), ((), ())),
        preferred_element_type=jnp.float32,
    )

    num = jnp.sum(x.astype(jnp.float32) * g[:, :5], axis=1)
    psel = num / g[:, 5]
    e = jnp.exp(_W * jnp.maximum(psel, 0.0))
    s = jnp.sum(e)

    @pl.when(t == 0)
    def _():
        out_ref[...] = jnp.zeros_like(out_ref)

    out_ref[...] += s

    @pl.when(t == nt - 1)
    def _():
        out_ref[...] *= _GAMMA


def kernel(xyz1, xyz2, nxyz2):
    b, n1, _ = xyz1.shape
    n2 = xyz2.shape[1]

    tn1 = min(512, n1)
    nt = n1 // tn1

    x_aug = jnp.concatenate(
        [xyz1, jnp.ones((b, n1, 2), jnp.float32)],
        axis=-1).astype(jnp.bfloat16)

    y_t = jnp.transpose(xyz2, (0, 2, 1))
    sq2 = jnp.sum(y_t * y_t, axis=1, keepdims=True)
    hi = sq2.astype(jnp.bfloat16)
    hi_f32 = jax.lax.optimization_barrier(hi).astype(jnp.float32)
    lo = (sq2 - hi_f32).astype(jnp.bfloat16)
    rhs = jnp.concatenate(
        [(-2.0 * y_t).astype(jnp.bfloat16), hi, lo], axis=1)

    c = jnp.sum(xyz2 * nxyz2, axis=-1, keepdims=True)
    ones = jnp.ones((b, n2, 1), jnp.float32)
    zeros = jnp.zeros((b, n2, 1), jnp.float32)
    nc = jnp.concatenate(
        [nxyz2, zeros, -c, ones, zeros, zeros],
        axis=-1).astype(jnp.bfloat16)

    sums = pl.pallas_call(
        functools.partial(_nn_kernel, nt=nt),
        grid=(b, nt),
        in_specs=[
            pl.BlockSpec((1, tn1, 5), lambda bi, ti: (bi, ti, 0)),
            pl.BlockSpec((1, 5, n2), lambda bi, ti: (bi, 0, 0)),
            pl.BlockSpec((1, n2, 8), lambda bi, ti: (bi, 0, 0)),
        ],
        out_specs=pl.BlockSpec((1, 8, 128), lambda bi, ti: (bi, 0, 0)),
        out_shape=jax.ShapeDtypeStruct((b, 8, 128), jnp.float32),
    )(x_aug, rhs, nc)

    return sums[:, 0, 0] / n1


# speedup vs baseline: 2.1826x; 2.1826x over previous
import functools

import jax
import jax.numpy as jnp
from jax.experimental import pallas as pl

_W = 5.0
_GAMMA = 0.02


def _nn_kernel(x_ref, rhs_ref, out_ref, *, n2, nt):
    t = pl.program_id(1)

    x = x_ref[0]
    rhs = rhs_ref[0]

    both = jax.lax.dot_general(
        x, rhs, (((1,), (0,)), ((), ())),
        preferred_element_type=jnp.float32,
    )
    d = both[:, :n2]
    p = both[:, n2:]

    m = jnp.min(d, axis=1, keepdims=True)
    psel = jnp.max(jnp.where(d == m, p, -jnp.inf), axis=1)
    e = jnp.exp(_W * jnp.maximum(psel, 0.0))
    s = jnp.sum(e)

    @pl.when(t == 0)
    def _():
        out_ref[...] = jnp.zeros_like(out_ref)

    out_ref[...] += s

    @pl.when(t == nt - 1)
    def _():
        out_ref[...] *= _GAMMA


def kernel(xyz1, xyz2, nxyz2):
    b, n1, _ = xyz1.shape
    n2 = xyz2.shape[1]

    tn1 = min(512, n1)
    nt = n1 // tn1

    x_aug = jnp.concatenate(
        [xyz1, jnp.ones((b, n1, 2), jnp.float32)],
        axis=-1).astype(jnp.bfloat16)

    y_t = jnp.transpose(xyz2, (0, 2, 1))
    n_t = jnp.transpose(nxyz2, (0, 2, 1))
    sq2 = jnp.sum(y_t * y_t, axis=1, keepdims=True)
    c = jnp.sum(y_t * n_t, axis=1, keepdims=True)
    hi = sq2.astype(jnp.bfloat16)
    hi_f32 = jax.lax.optimization_barrier(hi).astype(jnp.float32)
    lo = (sq2 - hi_f32).astype(jnp.bfloat16)
    zero = jnp.zeros_like(hi)
    rhs = jnp.concatenate([
        jnp.concatenate(
            [(-2.0 * y_t).astype(jnp.bfloat16), hi, lo], axis=1),
        jnp.concatenate(
            [n_t.astype(jnp.bfloat16), (-c).astype(jnp.bfloat16), zero],
            axis=1),
    ], axis=-1)

    sums = pl.pallas_call(
        functools.partial(_nn_kernel, n2=n2, nt=nt),
        grid=(b, nt),
        in_specs=[
            pl.BlockSpec((1, tn1, 5), lambda bi, ti: (bi, ti, 0)),
            pl.BlockSpec((1, 5, 2 * n2), lambda bi, ti: (bi, 0, 0)),
        ],
        out_specs=pl.BlockSpec((1, 8, 128), lambda bi, ti: (bi, 0, 0)),
        out_shape=jax.ShapeDtypeStruct((b, 8, 128), jnp.float32),
    )(x_aug, rhs)

    return sums[:, 0, 0] / n1


# speedup vs baseline: 2.3200x; 1.0629x over previous
import functools

import jax
import jax.numpy as jnp
from jax.experimental import pallas as pl

_W = 5.0
_GAMMA = 0.02


def _nn_kernel(x_ref, rhs_ref, out_ref, *, n2, nt):
    t = pl.program_id(1)

    x = x_ref[0]
    rhs = rhs_ref[0]

    both = jax.lax.dot_general(
        x, rhs, (((1,), (0,)), ((), ())),
        preferred_element_type=jnp.float32,
    )
    d = both[:, :n2]
    p = both[:, n2:]

    m = jnp.min(d, axis=1, keepdims=True)
    psel = jnp.max(jnp.where(d == m, p, -jnp.inf), axis=1)
    e = jnp.exp(_W * jnp.maximum(psel, 0.0))
    s = jnp.sum(e)

    @pl.when(t == 0)
    def _():
        out_ref[...] = jnp.zeros_like(out_ref)

    out_ref[...] += s

    @pl.when(t == nt - 1)
    def _():
        out_ref[...] *= _GAMMA


def kernel(xyz1, xyz2, nxyz2):
    b, n1, _ = xyz1.shape
    n2 = xyz2.shape[1]

    tn1 = min(1024, n1)
    nt = n1 // tn1

    x_aug = jnp.concatenate(
        [xyz1, jnp.ones((b, n1, 2), jnp.float32)],
        axis=-1).astype(jnp.bfloat16)

    y_t = jnp.transpose(xyz2, (0, 2, 1))
    n_t = jnp.transpose(nxyz2, (0, 2, 1))
    sq2 = jnp.sum(y_t * y_t, axis=1, keepdims=True)
    c = jnp.sum(y_t * n_t, axis=1, keepdims=True)
    hi = sq2.astype(jnp.bfloat16)
    hi_f32 = jax.lax.optimization_barrier(hi).astype(jnp.float32)
    lo = (sq2 - hi_f32).astype(jnp.bfloat16)
    zero = jnp.zeros_like(hi)
    rhs = jnp.concatenate([
        jnp.concatenate(
            [(-2.0 * y_t).astype(jnp.bfloat16), hi, lo], axis=1),
        jnp.concatenate(
            [n_t.astype(jnp.bfloat16), (-c).astype(jnp.bfloat16), zero],
            axis=1),
    ], axis=-1)

    sums = pl.pallas_call(
        functools.partial(_nn_kernel, n2=n2, nt=nt),
        grid=(b, nt),
        in_specs=[
            pl.BlockSpec((1, tn1, 5), lambda bi, ti: (bi, ti, 0)),
            pl.BlockSpec((1, 5, 2 * n2), lambda bi, ti: (bi, 0, 0)),
        ],
        out_specs=pl.BlockSpec((1, 8, 128), lambda bi, ti: (bi, 0, 0)),
        out_shape=jax.ShapeDtypeStruct((b, 8, 128), jnp.float32),
    )(x_aug, rhs)

    return sums[:, 0, 0] / n1


# speedup vs baseline: 2.3826x; 1.0270x over previous
import functools

import jax
import jax.numpy as jnp
from jax.experimental import pallas as pl

_W = 5.0
_GAMMA = 0.02


def _nn_kernel(x_ref, rhs_ref, out_ref, *, n2, nt):
    t = pl.program_id(1)

    x = x_ref[0]
    rhs = rhs_ref[0]

    both = jax.lax.dot_general(
        x, rhs, (((1,), (0,)), ((), ())),
        preferred_element_type=jnp.float32,
    )
    d = both[:, :n2]
    p = both[:, n2:]

    m = jnp.min(d, axis=1, keepdims=True)
    psel = jnp.max(jnp.where(d == m, p, -jnp.inf), axis=1)
    e = jnp.exp(_W * jnp.maximum(psel, 0.0))
    s = jnp.sum(e)

    @pl.when(t == 0)
    def _():
        out_ref[...] = jnp.zeros_like(out_ref)

    out_ref[...] += s

    @pl.when(t == nt - 1)
    def _():
        out_ref[...] *= _GAMMA


def kernel(xyz1, xyz2, nxyz2):
    b, n1, _ = xyz1.shape
    n2 = xyz2.shape[1]

    tn1 = min(2048, n1)
    nt = n1 // tn1

    x_aug = jnp.concatenate(
        [xyz1, jnp.ones((b, n1, 2), jnp.float32)],
        axis=-1).astype(jnp.bfloat16)

    y_t = jnp.transpose(xyz2, (0, 2, 1))
    n_t = jnp.transpose(nxyz2, (0, 2, 1))
    sq2 = jnp.sum(y_t * y_t, axis=1, keepdims=True)
    c = jnp.sum(y_t * n_t, axis=1, keepdims=True)
    hi = sq2.astype(jnp.bfloat16)
    hi_f32 = jax.lax.optimization_barrier(hi).astype(jnp.float32)
    lo = (sq2 - hi_f32).astype(jnp.bfloat16)
    zero = jnp.zeros_like(hi)
    rhs = jnp.concatenate([
        jnp.concatenate(
            [(-2.0 * y_t).astype(jnp.bfloat16), hi, lo], axis=1),
        jnp.concatenate(
            [n_t.astype(jnp.bfloat16), (-c).astype(jnp.bfloat16), zero],
            axis=1),
    ], axis=-1)

    sums = pl.pallas_call(
        functools.partial(_nn_kernel, n2=n2, nt=nt),
        grid=(b, nt),
        in_specs=[
            pl.BlockSpec((1, tn1, 5), lambda bi, ti: (bi, ti, 0)),
            pl.BlockSpec((1, 5, 2 * n2), lambda bi, ti: (bi, 0, 0)),
        ],
        out_specs=pl.BlockSpec((1, 8, 128), lambda bi, ti: (bi, 0, 0)),
        out_shape=jax.ShapeDtypeStruct((b, 8, 128), jnp.float32),
    )(x_aug, rhs)

    return sums[:, 0, 0] / n1
